# baseline (speedup 1.0000x reference)
import jax
import jax.numpy as jnp
from jax.experimental import pallas as pl
from jax.experimental.pallas import tpu as pltpu

_NE = 64
_CAP = 128
_T = 128


def _gate_kernel(ids_ref, valid_ref, combine_ref, dispatch_ref, counts_ref):
    g = pl.program_id(0)

    @pl.when(g == 0)
    def _():
        counts_ref[...] = jnp.zeros_like(counts_ref)

    ids = ids_ref[pl.ds(g * _T, _T)]
    valid = valid_ref[pl.ds(g * _T, _T)]

    e_iota = jax.lax.broadcasted_iota(jnp.int32, (_T, _NE), 1)
    mask1 = ((ids[:, None] == e_iota) & (valid[:, None] == 1)).astype(jnp.int32)

    r_iota = jax.lax.broadcasted_iota(jnp.int32, (_T, _T), 0)
    c_iota = jax.lax.broadcasted_iota(jnp.int32, (_T, _T), 1)
    tril = (r_iota >= c_iota).astype(jnp.float32)
    csum = jnp.dot(tril, mask1.astype(jnp.float32),
                   preferred_element_type=jnp.float32).astype(jnp.int32)
    loc = csum + counts_ref[...] - 1
    counts_ref[...] = counts_ref[...] + jnp.sum(mask1, axis=0, keepdims=True)

    kept = mask1 * (loc < _CAP).astype(jnp.int32)
    loc_s = jnp.sum(loc * kept, axis=1)
    kept_t = jnp.sum(kept, axis=1)

    target = jnp.where(kept_t > 0, ids * _CAP + loc_s, -1)[:, None]

    s_iota = jax.lax.broadcasted_iota(jnp.int32, (_T, _CAP), 1)
    for e in range(_NE):
        val_e = (s_iota + (e * _CAP)) == target
        combine_ref[:, e, :] = val_e.astype(jnp.float32)
        dispatch_ref[:, e, :] = val_e


def kernel(input, mask, domain_ids):
    n_tokens = input.shape[0]
    grid = n_tokens // _T
    ids = domain_ids.astype(jnp.int32)
    valid = jnp.logical_not(mask).astype(jnp.int32)

    combine, dispatch = pl.pallas_call(
        _gate_kernel,
        grid=(grid,),
        in_specs=[
            pl.BlockSpec((n_tokens,), lambda g: (0,)),
            pl.BlockSpec((n_tokens,), lambda g: (0,)),
        ],
        out_specs=[
            pl.BlockSpec((_T, _NE, _CAP), lambda g: (g, 0, 0)),
            pl.BlockSpec((_T, _NE, _CAP), lambda g: (g, 0, 0)),
        ],
        out_shape=[
            jax.ShapeDtypeStruct((n_tokens, _NE, _CAP), jnp.float32),
            jax.ShapeDtypeStruct((n_tokens, _NE, _CAP), jnp.bool_),
        ],
        scratch_shapes=[pltpu.VMEM((1, _NE), jnp.int32)],
    )(ids, valid)

    l_aux = jnp.zeros((), dtype=jnp.float32)
    return (l_aux, combine, dispatch)


# speedup vs baseline: 2.6727x; 2.6727x over previous
import jax
import jax.numpy as jnp
from jax.experimental import pallas as pl
from jax.experimental.pallas import tpu as pltpu

_NE = 64
_CAP = 128
_T = 128


def _gate_kernel(ids_ref, valid_ref, combine_ref, dispatch_ref, counts_ref):
    g = pl.program_id(0)

    @pl.when(g == 0)
    def _():
        for e in range(_NE):
            counts_ref[e] = 0

    e_iota = jax.lax.broadcasted_iota(jnp.int32, (_NE, _CAP), 0)
    c_iota = jax.lax.broadcasted_iota(jnp.int32, (_NE, _CAP), 1)
    flat_iota = e_iota * _CAP + c_iota

    def body(i, _):
        t = g * _T + i
        e = ids_ref[t]
        v = valid_ref[t]
        cnt = counts_ref[e]
        counts_ref[e] = cnt + v
        kept = (v == 1) & (cnt < _CAP)
        tgt = jnp.where(kept, e * _CAP + cnt, -1)
        slab = flat_iota == tgt
        combine_ref[i] = slab.astype(jnp.float32)
        dispatch_ref[i] = slab
        return 0

    jax.lax.fori_loop(0, _T, body, 0)


def kernel(input, mask, domain_ids):
    n_tokens = input.shape[0]
    grid = n_tokens // _T
    ids = domain_ids.astype(jnp.int32)
    valid = jnp.logical_not(mask).astype(jnp.int32)

    combine, dispatch = pl.pallas_call(
        _gate_kernel,
        grid=(grid,),
        in_specs=[
            pl.BlockSpec(memory_space=pltpu.SMEM),
            pl.BlockSpec(memory_space=pltpu.SMEM),
        ],
        out_specs=[
            pl.BlockSpec((_T, _NE, _CAP), lambda g: (g, 0, 0)),
            pl.BlockSpec((_T, _NE, _CAP), lambda g: (g, 0, 0)),
        ],
        out_shape=[
            jax.ShapeDtypeStruct((n_tokens, _NE, _CAP), jnp.float32),
            jax.ShapeDtypeStruct((n_tokens, _NE, _CAP), jnp.bool_),
        ],
        scratch_shapes=[pltpu.SMEM((_NE,), jnp.int32)],
    )(ids, valid)

    l_aux = jnp.zeros((), dtype=jnp.float32)
    return (l_aux, combine, dispatch)
